# baseline (speedup 1.0000x reference)
import functools

import jax
import jax.numpy as jnp
from jax import lax
from jax.experimental import pallas as pl
from jax.experimental.pallas import tpu as pltpu
from jax.experimental.pallas import tpu_sc as plsc

N = 10000
E = 320000
D = 128
NC = 2
NS = 16
NW = NC * NS
NPAD = 10240
NPADP = NPAD + 16
DUMMY = NPAD
CHD = 1024
CH = 2048
EPT = 10240
PADE = EPT * NW
NCH_DEG = EPT // CHD
NCH_AGG = PADE // CH
CPT = D // NW

_mesh = plsc.VectorSubcoreMesh(core_axis_name="c", subcore_axis_name="s")
_sc_params = pltpu.CompilerParams(needs_layout_passes=False)


@functools.partial(
    pl.kernel,
    mesh=_mesh,
    compiler_params=_sc_params,
    out_type=jax.ShapeDtypeStruct((NW, NPADP), jnp.float32),
    scratch_types=[
        pltpu.VMEM((CHD,), jnp.int32),
        pltpu.VMEM((CHD,), jnp.int32),
        pltpu.VMEM((NPADP,), jnp.float32),
    ],
)
def _deg_sc(src_h, dst_h, zeros_h, out_h, sidx_v, didx_v, acc_v):
    cid = lax.axis_index("c")
    sid = lax.axis_index("s")
    wid = sid * NC + cid

    pltpu.sync_copy(zeros_h, acc_v)
    ones = jnp.full((16,), 1.0, jnp.float32)

    def body(c, carry):
        base = wid * EPT + c * CHD
        pltpu.sync_copy(src_h.at[pl.ds(base, CHD)], sidx_v)
        pltpu.sync_copy(dst_h.at[pl.ds(base, CHD)], didx_v)
        for j in range(CHD // 16):
            s = sidx_v[pl.ds(j * 16, 16)]
            d = didx_v[pl.ds(j * 16, 16)]
            dm = jnp.where(s == d, DUMMY, d)
            plsc.addupdate_scatter(acc_v, [dm], ones)
        return carry

    lax.fori_loop(0, NCH_DEG, body, 0)
    pltpu.sync_copy(acc_v, out_h.at[wid])


@functools.partial(
    pl.kernel,
    mesh=_mesh,
    compiler_params=_sc_params,
    out_type=jax.ShapeDtypeStruct((D, NPAD), jnp.float32),
    scratch_types=[
        [pltpu.VMEM((CH,), jnp.int32) for _ in range(2)],
        [pltpu.VMEM((CH,), jnp.int32) for _ in range(2)],
        [pltpu.VMEM((NPAD,), jnp.float32) for _ in range(CPT)],
        [pltpu.VMEM((NPADP,), jnp.float32) for _ in range(CPT)],
        [pltpu.SemaphoreType.DMA for _ in range(2)],
    ],
)
def _agg_sc(src_h, dst_h, zeros_h, ht_h, out_h, sidx, didx, hcol, acc, sem):
    cid = lax.axis_index("c")
    sid = lax.axis_index("s")
    wid = sid * NC + cid

    for k in range(CPT):
        pltpu.sync_copy(ht_h.at[wid * CPT + k], hcol[k])
        pltpu.sync_copy(zeros_h, acc[k])

    def start(b, c):
        pltpu.make_async_copy(
            src_h.at[pl.ds(c * CH, CH)], sidx[b], sem[b]).start()
        pltpu.make_async_copy(
            dst_h.at[pl.ds(c * CH, CH)], didx[b], sem[b]).start()

    def wait(b, c):
        pltpu.make_async_copy(
            src_h.at[pl.ds(c * CH, CH)], sidx[b], sem[b]).wait()
        pltpu.make_async_copy(
            dst_h.at[pl.ds(c * CH, CH)], didx[b], sem[b]).wait()

    def process(b):
        for j in range(CH // 16):
            s = sidx[b][pl.ds(j * 16, 16)]
            d = didx[b][pl.ds(j * 16, 16)]
            dm = jnp.where(s == d, DUMMY, d)
            for k in range(CPT):
                vals = plsc.load_gather(hcol[k], [s])
                plsc.addupdate_scatter(acc[k], [dm], vals)

    start(0, 0)
    start(1, 1)

    def body(g, carry):
        c = g * 2
        wait(0, c)
        process(0)
        start(0, c + 2)
        wait(1, c + 1)
        process(1)
        start(1, c + 3)
        return carry

    lax.fori_loop(0, NCH_AGG // 2 - 1, body, 0)
    wait(0, NCH_AGG - 2)
    process(0)
    wait(1, NCH_AGG - 1)
    process(1)

    for k in range(CPT):
        pltpu.sync_copy(acc[k].at[pl.ds(0, NPAD)], out_h.at[wid * CPT + k])


R = 1280


def _s2_body(f_ref, wt_ref, cnt_ref, h_ref, ht_ref, r_ref):
    cnt = jnp.sum(cnt_ref[...], axis=0)
    r = jax.lax.rsqrt(cnt + 1.0)
    z = jnp.dot(f_ref[...], wt_ref[...], preferred_element_type=jnp.float32)
    h = z * r
    h_ref[...] = h
    ht_ref[...] = h.T
    r_ref[...] = r


_stage2 = pl.pallas_call(
    _s2_body,
    grid=(NPAD // R,),
    in_specs=[
        pl.BlockSpec((R, D), lambda i: (i, 0)),
        pl.BlockSpec((D, D), lambda i: (0, 0)),
        pl.BlockSpec((NW, R, 1), lambda i: (0, i, 0)),
    ],
    out_specs=[
        pl.BlockSpec((R, D), lambda i: (i, 0)),
        pl.BlockSpec((D, R), lambda i: (0, i)),
        pl.BlockSpec((R, 1), lambda i: (i, 0)),
    ],
    out_shape=[
        jax.ShapeDtypeStruct((NPAD, D), jnp.float32),
        jax.ShapeDtypeStruct((D, NPAD), jnp.float32),
        jax.ShapeDtypeStruct((NPAD, 1), jnp.float32),
    ],
)


def _s4_body(aggt_ref, h_ref, r_ref, b_ref, o_ref):
    s = aggt_ref[...].T + h_ref[...]
    o_ref[...] = jnp.maximum(s * r_ref[...] + b_ref[...], 0.0)


_stage4 = pl.pallas_call(
    _s4_body,
    grid=(NPAD // R,),
    in_specs=[
        pl.BlockSpec((D, R), lambda i: (0, i)),
        pl.BlockSpec((R, D), lambda i: (i, 0)),
        pl.BlockSpec((R, 1), lambda i: (i, 0)),
        pl.BlockSpec((1, D), lambda i: (0, 0)),
    ],
    out_specs=pl.BlockSpec((R, D), lambda i: (i, 0)),
    out_shape=jax.ShapeDtypeStruct((NPAD, D), jnp.float32),
)


def kernel(feature, edge_index, W, bias):
    src = edge_index[0].astype(jnp.int32)
    dst = edge_index[1].astype(jnp.int32)
    pad = PADE - E
    src = jnp.concatenate([src, jnp.zeros((pad,), jnp.int32)])
    dst = jnp.concatenate([dst, jnp.zeros((pad,), jnp.int32)])
    featp = jnp.concatenate(
        [feature, jnp.zeros((NPAD - N, D), jnp.float32)])
    zerosp = jnp.zeros((NPADP,), jnp.float32)
    wt = W.T
    bias2 = bias.reshape(1, D)

    degp = _deg_sc(src, dst, zerosp)
    cnt = degp[:, :NPAD, None]
    h, ht, r = _stage2(featp, wt, cnt)
    aggt = _agg_sc(src, dst, zerosp, ht)
    out = _stage4(aggt, h, r, bias2)
    return out[:N]


# speedup vs baseline: 1.0870x; 1.0870x over previous
import functools

import jax
import jax.numpy as jnp
from jax import lax
from jax.experimental import pallas as pl
from jax.experimental.pallas import tpu as pltpu
from jax.experimental.pallas import tpu_sc as plsc

N = 10000
E = 320000
D = 128
NC = 2
NS = 16
NW = NC * NS
NPAD = 10240
NPADP = NPAD + 16
DUMMY = NPAD
CHD = 1024
CH = 1024
EPT = 10240
PADE = EPT * NW
NCH_DEG = EPT // CHD
NCH_AGG = PADE // CH
CPT = D // NW

_mesh = plsc.VectorSubcoreMesh(core_axis_name="c", subcore_axis_name="s")
_sc_params = pltpu.CompilerParams(needs_layout_passes=False)


@functools.partial(
    pl.kernel,
    mesh=_mesh,
    compiler_params=_sc_params,
    out_type=jax.ShapeDtypeStruct((NW, NPADP), jnp.float32),
    scratch_types=[
        pltpu.VMEM((CHD,), jnp.int32),
        pltpu.VMEM((CHD,), jnp.int32),
        pltpu.VMEM((NPADP,), jnp.float32),
    ],
)
def _deg_sc(src_h, dst_h, zeros_h, out_h, sidx_v, didx_v, acc_v):
    cid = lax.axis_index("c")
    sid = lax.axis_index("s")
    wid = sid * NC + cid

    pltpu.sync_copy(zeros_h, acc_v)
    ones = jnp.full((16,), 1.0, jnp.float32)

    def body(c, carry):
        base = wid * EPT + c * CHD
        pltpu.sync_copy(src_h.at[pl.ds(base, CHD)], sidx_v)
        pltpu.sync_copy(dst_h.at[pl.ds(base, CHD)], didx_v)
        for j in range(CHD // 16):
            s = sidx_v[pl.ds(j * 16, 16)]
            d = didx_v[pl.ds(j * 16, 16)]
            dm = jnp.where(s == d, DUMMY, d)
            plsc.addupdate_scatter(acc_v, [dm], ones)
        return carry

    lax.fori_loop(0, NCH_DEG, body, 0)
    pltpu.sync_copy(acc_v, out_h.at[wid])


@functools.partial(
    pl.kernel,
    mesh=_mesh,
    compiler_params=_sc_params,
    out_type=jax.ShapeDtypeStruct((D, NPAD), jnp.float32),
    scratch_types=[
        [pltpu.VMEM((CH,), jnp.int32) for _ in range(2)],
        [pltpu.VMEM((CH,), jnp.int32) for _ in range(2)],
        [pltpu.VMEM((NPAD,), jnp.float32) for _ in range(CPT)],
        [pltpu.VMEM((NPADP,), jnp.float32) for _ in range(CPT)],
        [pltpu.SemaphoreType.DMA for _ in range(2)],
    ],
)
def _agg_sc(src_h, dst_h, zeros_h, ht_h, out_h, sidx, didx, hcol, acc, sem):
    cid = lax.axis_index("c")
    sid = lax.axis_index("s")
    wid = sid * NC + cid

    for k in range(CPT):
        pltpu.sync_copy(ht_h.at[wid * CPT + k], hcol[k])
        pltpu.sync_copy(zeros_h, acc[k])

    def start(b, c):
        pltpu.make_async_copy(
            src_h.at[pl.ds(c * CH, CH)], sidx[b], sem[b]).start()
        pltpu.make_async_copy(
            dst_h.at[pl.ds(c * CH, CH)], didx[b], sem[b]).start()

    def wait(b, c):
        pltpu.make_async_copy(
            src_h.at[pl.ds(c * CH, CH)], sidx[b], sem[b]).wait()
        pltpu.make_async_copy(
            dst_h.at[pl.ds(c * CH, CH)], didx[b], sem[b]).wait()

    def process(b):
        def pbody(j, carry):
            s = sidx[b][pl.ds(j * 16, 16)]
            d = didx[b][pl.ds(j * 16, 16)]
            dm = jnp.where(s == d, DUMMY, d)
            for k in range(CPT):
                vals = plsc.load_gather(hcol[k], [s])
                plsc.addupdate_scatter(acc[k], [dm], vals)
            return carry
        lax.fori_loop(0, CH // 16, pbody, 0)

    start(0, 0)
    start(1, 1)

    def body(g, carry):
        c = g * 2
        wait(0, c)
        process(0)
        start(0, c + 2)
        wait(1, c + 1)
        process(1)
        start(1, c + 3)
        return carry

    lax.fori_loop(0, NCH_AGG // 2 - 1, body, 0)
    wait(0, NCH_AGG - 2)
    process(0)
    wait(1, NCH_AGG - 1)
    process(1)

    for k in range(CPT):
        pltpu.sync_copy(acc[k].at[pl.ds(0, NPAD)], out_h.at[wid * CPT + k])


R = 1280


def _s2_body(f_ref, wt_ref, cnt_ref, h_ref, ht_ref, r_ref):
    cnt = jnp.sum(cnt_ref[...], axis=0)
    r = jax.lax.rsqrt(cnt + 1.0)
    z = jnp.dot(f_ref[...], wt_ref[...], preferred_element_type=jnp.float32)
    h = z * r
    h_ref[...] = h
    ht_ref[...] = h.T
    r_ref[...] = r


_stage2 = pl.pallas_call(
    _s2_body,
    grid=(NPAD // R,),
    in_specs=[
        pl.BlockSpec((R, D), lambda i: (i, 0)),
        pl.BlockSpec((D, D), lambda i: (0, 0)),
        pl.BlockSpec((NW, R, 1), lambda i: (0, i, 0)),
    ],
    out_specs=[
        pl.BlockSpec((R, D), lambda i: (i, 0)),
        pl.BlockSpec((D, R), lambda i: (0, i)),
        pl.BlockSpec((R, 1), lambda i: (i, 0)),
    ],
    out_shape=[
        jax.ShapeDtypeStruct((NPAD, D), jnp.float32),
        jax.ShapeDtypeStruct((D, NPAD), jnp.float32),
        jax.ShapeDtypeStruct((NPAD, 1), jnp.float32),
    ],
)


def _s4_body(aggt_ref, h_ref, r_ref, b_ref, o_ref):
    s = aggt_ref[...].T + h_ref[...]
    o_ref[...] = jnp.maximum(s * r_ref[...] + b_ref[...], 0.0)


_stage4 = pl.pallas_call(
    _s4_body,
    grid=(NPAD // R,),
    in_specs=[
        pl.BlockSpec((D, R), lambda i: (0, i)),
        pl.BlockSpec((R, D), lambda i: (i, 0)),
        pl.BlockSpec((R, 1), lambda i: (i, 0)),
        pl.BlockSpec((1, D), lambda i: (0, 0)),
    ],
    out_specs=pl.BlockSpec((R, D), lambda i: (i, 0)),
    out_shape=jax.ShapeDtypeStruct((NPAD, D), jnp.float32),
)


def kernel(feature, edge_index, W, bias):
    src = edge_index[0].astype(jnp.int32)
    dst = edge_index[1].astype(jnp.int32)
    pad = PADE - E
    src = jnp.concatenate([src, jnp.zeros((pad,), jnp.int32)])
    dst = jnp.concatenate([dst, jnp.zeros((pad,), jnp.int32)])
    featp = jnp.concatenate(
        [feature, jnp.zeros((NPAD - N, D), jnp.float32)])
    zerosp = jnp.zeros((NPADP,), jnp.float32)
    wt = W.T
    bias2 = bias.reshape(1, D)

    degp = _deg_sc(src, dst, zerosp)
    cnt = degp[:, :NPAD, None]
    h, ht, r = _stage2(featp, wt, cnt)
    aggt = _agg_sc(src, dst, zerosp, ht)
    out = _stage4(aggt, h, r, bias2)
    return out[:N]


# speedup vs baseline: 1.0874x; 1.0003x over previous
import functools

import jax
import jax.numpy as jnp
from jax import lax
from jax.experimental import pallas as pl
from jax.experimental.pallas import tpu as pltpu
from jax.experimental.pallas import tpu_sc as plsc

N = 10000
E = 320000
D = 128
NC = 2
NS = 16
NW = NC * NS
NPAD = 10240
NPADP = NPAD + 16
DUMMY = NPAD
CHD = 1024
CH = 1024
EPT = 10240
PADE = EPT * NW
NCH_DEG = EPT // CHD
NCH_AGG = PADE // CH
CPT = D // NW

_mesh = plsc.VectorSubcoreMesh(core_axis_name="c", subcore_axis_name="s")
_sc_params = pltpu.CompilerParams(needs_layout_passes=False)


@functools.partial(
    pl.kernel,
    mesh=_mesh,
    compiler_params=_sc_params,
    out_type=jax.ShapeDtypeStruct((NW, NPADP), jnp.float32),
    scratch_types=[
        pltpu.VMEM((CHD,), jnp.int32),
        pltpu.VMEM((CHD,), jnp.int32),
        pltpu.VMEM((NPADP,), jnp.float32),
    ],
)
def _deg_sc(src_h, dst_h, zeros_h, out_h, sidx_v, didx_v, acc_v):
    cid = lax.axis_index("c")
    sid = lax.axis_index("s")
    wid = sid * NC + cid

    pltpu.sync_copy(zeros_h, acc_v)
    ones = jnp.full((16,), 1.0, jnp.float32)

    def body(c, carry):
        base = wid * EPT + c * CHD
        pltpu.sync_copy(src_h.at[pl.ds(base, CHD)], sidx_v)
        pltpu.sync_copy(dst_h.at[pl.ds(base, CHD)], didx_v)
        for j in range(CHD // 16):
            s = sidx_v[pl.ds(j * 16, 16)]
            d = didx_v[pl.ds(j * 16, 16)]
            dm = jnp.where(s == d, DUMMY, d)
            plsc.addupdate_scatter(acc_v, [dm], ones)
        return carry

    lax.fori_loop(0, NCH_DEG, body, 0)
    pltpu.sync_copy(acc_v, out_h.at[wid])


@functools.partial(
    pl.kernel,
    mesh=_mesh,
    compiler_params=_sc_params,
    out_type=jax.ShapeDtypeStruct((D, NPAD), jnp.float32),
    scratch_types=[
        [pltpu.VMEM((CH,), jnp.int32) for _ in range(2)],
        [pltpu.VMEM((CH,), jnp.int32) for _ in range(2)],
        [pltpu.VMEM((NPAD,), jnp.float32) for _ in range(CPT)],
        [pltpu.VMEM((NPADP,), jnp.float32) for _ in range(CPT)],
        [pltpu.SemaphoreType.DMA for _ in range(2)],
        [pltpu.SemaphoreType.DMA for _ in range(2)],
    ],
)
def _agg_sc(src_h, dst_h, zeros_h, ht_h, out_h, sidx, didx, hcol, acc,
            sems, semd):
    cid = lax.axis_index("c")
    sid = lax.axis_index("s")
    wid = sid * NC + cid

    for k in range(CPT):
        pltpu.sync_copy(ht_h.at[wid * CPT + k], hcol[k])
        pltpu.sync_copy(zeros_h, acc[k])

    def start(b, c):
        pltpu.make_async_copy(
            src_h.at[pl.ds(c * CH, CH)], sidx[b], sems[b]).start()
        pltpu.make_async_copy(
            dst_h.at[pl.ds(c * CH, CH)], didx[b], semd[b]).start()

    def wait(b, c):
        pltpu.make_async_copy(
            src_h.at[pl.ds(c * CH, CH)], sidx[b], sems[b]).wait()
        pltpu.make_async_copy(
            dst_h.at[pl.ds(c * CH, CH)], didx[b], semd[b]).wait()

    def process(b):
        def pbody(j, carry):
            s = sidx[b][pl.ds(j * 16, 16)]
            d = didx[b][pl.ds(j * 16, 16)]
            dm = jnp.where(s == d, DUMMY, d)
            for k in range(CPT):
                vals = plsc.load_gather(hcol[k], [s])
                plsc.addupdate_scatter(acc[k], [dm], vals)
            return carry
        lax.fori_loop(0, CH // 16, pbody, 0)

    start(0, 0)
    start(1, 1)

    def body(g, carry):
        c = g * 2
        wait(0, c)
        process(0)
        start(0, c + 2)
        wait(1, c + 1)
        process(1)
        start(1, c + 3)
        return carry

    lax.fori_loop(0, NCH_AGG // 2 - 1, body, 0)
    wait(0, NCH_AGG - 2)
    process(0)
    wait(1, NCH_AGG - 1)
    process(1)

    for k in range(CPT):
        pltpu.sync_copy(acc[k].at[pl.ds(0, NPAD)], out_h.at[wid * CPT + k])


R = 1280


def _s2_body(f_ref, wt_ref, cnt_ref, h_ref, ht_ref, r_ref):
    cnt = jnp.sum(cnt_ref[...], axis=0)
    r = jax.lax.rsqrt(cnt + 1.0)
    z = jnp.dot(f_ref[...], wt_ref[...], preferred_element_type=jnp.float32)
    h = z * r
    h_ref[...] = h
    ht_ref[...] = h.T
    r_ref[...] = r


_stage2 = pl.pallas_call(
    _s2_body,
    grid=(NPAD // R,),
    in_specs=[
        pl.BlockSpec((R, D), lambda i: (i, 0)),
        pl.BlockSpec((D, D), lambda i: (0, 0)),
        pl.BlockSpec((NW, R, 1), lambda i: (0, i, 0)),
    ],
    out_specs=[
        pl.BlockSpec((R, D), lambda i: (i, 0)),
        pl.BlockSpec((D, R), lambda i: (0, i)),
        pl.BlockSpec((R, 1), lambda i: (i, 0)),
    ],
    out_shape=[
        jax.ShapeDtypeStruct((NPAD, D), jnp.float32),
        jax.ShapeDtypeStruct((D, NPAD), jnp.float32),
        jax.ShapeDtypeStruct((NPAD, 1), jnp.float32),
    ],
)


def _s4_body(aggt_ref, h_ref, r_ref, b_ref, o_ref):
    s = aggt_ref[...].T + h_ref[...]
    o_ref[...] = jnp.maximum(s * r_ref[...] + b_ref[...], 0.0)


_stage4 = pl.pallas_call(
    _s4_body,
    grid=(NPAD // R,),
    in_specs=[
        pl.BlockSpec((D, R), lambda i: (0, i)),
        pl.BlockSpec((R, D), lambda i: (i, 0)),
        pl.BlockSpec((R, 1), lambda i: (i, 0)),
        pl.BlockSpec((1, D), lambda i: (0, 0)),
    ],
    out_specs=pl.BlockSpec((R, D), lambda i: (i, 0)),
    out_shape=jax.ShapeDtypeStruct((NPAD, D), jnp.float32),
)


def kernel(feature, edge_index, W, bias):
    src = edge_index[0].astype(jnp.int32)
    dst = edge_index[1].astype(jnp.int32)
    pad = PADE - E
    src = jnp.concatenate([src, jnp.zeros((pad,), jnp.int32)])
    dst = jnp.concatenate([dst, jnp.zeros((pad,), jnp.int32)])
    featp = jnp.concatenate(
        [feature, jnp.zeros((NPAD - N, D), jnp.float32)])
    zerosp = jnp.zeros((NPADP,), jnp.float32)
    wt = W.T
    bias2 = bias.reshape(1, D)

    degp = _deg_sc(src, dst, zerosp)
    cnt = degp[:, :NPAD, None]
    h, ht, r = _stage2(featp, wt, cnt)
    aggt = _agg_sc(src, dst, zerosp, ht)
    out = _stage4(aggt, h, r, bias2)
    return out[:N]
